# baseline (speedup 1.0000x reference)
import jax
import jax.numpy as jnp
from jax.experimental import pallas as pl

VOCAB = 1000
CTX = 50
BB = 8


def _onehot_block(ids_ref, out_ref):
    ids = ids_ref[...]
    d = VOCAB + CTX
    c_iota = jax.lax.broadcasted_iota(jnp.int32, (BB, CTX, d), 2)
    n_iota = jax.lax.broadcasted_iota(jnp.int32, (BB, CTX, d), 1)
    hit = (c_iota == ids[:, :, None]) | (c_iota == n_iota + VOCAB)
    out_ref[...] = hit.astype(jnp.float32)


def kernel(token_ids):
    b, n = token_ids.shape
    d = VOCAB + CTX
    grid = (b // BB,)
    return pl.pallas_call(
        _onehot_block,
        grid=grid,
        in_specs=[pl.BlockSpec((BB, n), lambda i: (i, 0))],
        out_specs=pl.BlockSpec((BB, n, d), lambda i: (i, 0, 0)),
        out_shape=jax.ShapeDtypeStruct((b, n, d), jnp.float32),
    )(token_ids)


# speedup vs baseline: 1.0983x; 1.0983x over previous
import jax
import jax.numpy as jnp
from jax.experimental import pallas as pl
from jax.experimental.pallas import tpu as pltpu

VOCAB = 1000
CTX = 50
BB = 16
NBUF = 6


def _onehot_body(ids_ref, out_ref, scratch, sems):
    d = VOCAB + CTX
    i = pl.program_id(0)
    steps = pl.num_programs(0)
    slot = jax.lax.rem(i, NBUF)

    @pl.when(i >= NBUF)
    def _wait_prev():
        pltpu.make_async_copy(
            scratch.at[slot],
            out_ref.at[pl.ds((i - NBUF) * BB, BB)],
            sems.at[slot],
        ).wait()

    ids = ids_ref[...]
    c_iota = jax.lax.broadcasted_iota(jnp.int32, (BB, CTX, d), 2)
    n_iota = jax.lax.broadcasted_iota(jnp.int32, (BB, CTX, d), 1)
    hit = (c_iota == ids[:, :, None]) | (c_iota == n_iota + VOCAB)
    scratch[slot] = hit.astype(jnp.float32)

    pltpu.make_async_copy(
        scratch.at[slot],
        out_ref.at[pl.ds(i * BB, BB)],
        sems.at[slot],
    ).start()

    @pl.when(i == steps - 1)
    def _drain():
        for k in range(NBUF):
            step = i - (NBUF - 1) + k
            s = jax.lax.rem(step, NBUF)
            pltpu.make_async_copy(
                scratch.at[s],
                out_ref.at[pl.ds(step * BB, BB)],
                sems.at[s],
            ).wait()


def kernel(token_ids):
    b, n = token_ids.shape
    d = VOCAB + CTX
    steps = b // BB
    return pl.pallas_call(
        _onehot_body,
        grid=(steps,),
        in_specs=[pl.BlockSpec((BB, n), lambda i: (i, 0))],
        out_specs=pl.BlockSpec(memory_space=pltpu.MemorySpace.HBM),
        out_shape=jax.ShapeDtypeStruct((b, n, d), jnp.float32),
        scratch_shapes=[
            pltpu.VMEM((NBUF, BB, n, d), jnp.float32),
            pltpu.SemaphoreType.DMA((NBUF,)),
        ],
        compiler_params=pltpu.CompilerParams(
            dimension_semantics=("arbitrary",),
        ),
    )(token_ids)


# speedup vs baseline: 1.1025x; 1.0038x over previous
import jax
import jax.numpy as jnp
from jax.experimental import pallas as pl
from jax.experimental.pallas import tpu as pltpu

VOCAB = 1000
CTX = 50
BB = 16
NBUF = 4


def _onehot_body(ids_ref, out_ref, *scratch_and_sems):
    scratches = scratch_and_sems[:NBUF]
    sems = scratch_and_sems[NBUF]
    d = VOCAB + CTX
    i = pl.program_id(0)
    steps = pl.num_programs(0)
    slot = jax.lax.rem(i, NBUF)

    ids = ids_ref[...]
    c_iota = jax.lax.broadcasted_iota(jnp.int32, (BB, CTX, d), 2)
    n_iota = jax.lax.broadcasted_iota(jnp.int32, (BB, CTX, d), 1)
    block = ((c_iota == ids[:, :, None]) | (c_iota == n_iota + VOCAB)).astype(
        jnp.float32
    )

    for k in range(NBUF):
        @pl.when(slot == k)
        def _go(k=k):
            @pl.when(i >= NBUF)
            def _wait_prev():
                pltpu.make_async_copy(
                    scratches[k],
                    out_ref.at[pl.ds((i - NBUF) * BB, BB)],
                    sems.at[k],
                ).wait()

            scratches[k][...] = block
            pltpu.make_async_copy(
                scratches[k],
                out_ref.at[pl.ds(i * BB, BB)],
                sems.at[k],
            ).start()

    @pl.when(i == steps - 1)
    def _drain():
        for k in range(NBUF):
            step = i - (NBUF - 1) + k
            s = jax.lax.rem(step, NBUF)
            for kk in range(NBUF):
                @pl.when(s == kk)
                def _w(kk=kk, step=step):
                    pltpu.make_async_copy(
                        scratches[kk],
                        out_ref.at[pl.ds(step * BB, BB)],
                        sems.at[kk],
                    ).wait()


def kernel(token_ids):
    b, n = token_ids.shape
    d = VOCAB + CTX
    steps = b // BB
    return pl.pallas_call(
        _onehot_body,
        grid=(steps,),
        in_specs=[pl.BlockSpec((BB, n), lambda i: (i, 0))],
        out_specs=pl.BlockSpec(memory_space=pltpu.MemorySpace.HBM),
        out_shape=jax.ShapeDtypeStruct((b, n, d), jnp.float32),
        scratch_shapes=[pltpu.VMEM((BB, n, d), jnp.float32) for _ in range(NBUF)]
        + [pltpu.SemaphoreType.DMA((NBUF,))],
        compiler_params=pltpu.CompilerParams(
            dimension_semantics=("arbitrary",),
        ),
    )(token_ids)
